# baseline (speedup 1.0000x reference)
import functools

import jax
import jax.numpy as jnp
from jax import lax
from jax.experimental import pallas as pl
from jax.experimental.pallas import tpu as pltpu
from jax.experimental.pallas import tpu_sc as plsc

S = 2048
D = 768
E = 8
H = 3072
EPS = 1e-6
N_TOT = float(S * E)

GRID_L = 2
S_BLK = S // GRID_L
GRID_H = 2
H_BLK = H // GRID_H
GRID_Z = 4


def _gate_logits_body(x_ref, wg_ref, bg_ref, out_ref):
    h = jnp.dot(x_ref[...], wg_ref[...], preferred_element_type=jnp.float32)
    out_ref[...] = (h + bg_ref[...][None, :]).T


def _gate_logits(x, w_gate, b_gate_row):
    return pl.pallas_call(
        _gate_logits_body,
        grid=(GRID_L,),
        in_specs=[
            pl.BlockSpec((S_BLK, D), lambda i: (i, 0)),
            pl.BlockSpec((D, E), lambda i: (0, 0)),
            pl.BlockSpec((E,), lambda i: (0,)),
        ],
        out_specs=pl.BlockSpec((E, S_BLK), lambda i: (0, i)),
        out_shape=jax.ShapeDtypeStruct((E, S), jnp.float32),
    )(x, w_gate, b_gate_row)


@functools.lru_cache(maxsize=1)
def _make_sc_routing():
    mesh = plsc.VectorSubcoreMesh(core_axis_name="c", subcore_axis_name="s")
    return functools.partial(
        pl.kernel,
        out_type=(
            jax.ShapeDtypeStruct((16,), jnp.float32),
            jax.ShapeDtypeStruct((16,), jnp.float32),
        ),
        mesh=mesh,
        scratch_types=[
            pltpu.VMEM((E, S), jnp.float32),
            pltpu.VMEM((16,), jnp.float32),
            pltpu.VMEM((16,), jnp.float32),
        ],
    )(_sc_routing_body)


def _sc_routing_body(logits_hbm, gate_hbm, loss_hbm, lg_v, gout_v, loss_v):
    cid = lax.axis_index("c")
    sid = lax.axis_index("s")

    @pl.when((cid == 0) & (sid == 0))
    def _():
        pltpu.sync_copy(logits_hbm, lg_v)

        def body(i, cnts):
            off = i * 16
            vs = [lg_v[e, pl.ds(off, 16)] for e in range(E)]
            mx = vs[0]
            for e in range(1, E):
                mx = jnp.maximum(mx, vs[e])
            return tuple(
                cnts[e] + jnp.where(vs[e] >= mx, 1.0, 0.0) for e in range(E)
            )

        init = tuple(jnp.zeros((16,), jnp.float32) for _ in range(E))
        cnts = lax.fori_loop(0, S // 16, body, init)

        lane = lax.iota(jnp.int32, 16)
        member = jnp.zeros((16,), jnp.float32)
        for e in range(E):
            tot = cnts[e][0]
            for l in range(1, 16):
                tot = tot + cnts[e][l]
            flag = jnp.where(tot > 0.0, 1.0, 0.0)
            member = member + jnp.where(lane == e, flag, 0.0)

        v0 = [lg_v[e, pl.ds(0, 16)] for e in range(E)]
        mx = v0[0]
        for e in range(1, E):
            mx = jnp.maximum(mx, v0[e])
        den = jnp.zeros((16,), jnp.float32)
        for e in range(E):
            den = den + jnp.exp(v0[e] - mx)
        g = jnp.exp(v0[0] - mx) / den

        masked = member * g
        gate = masked / (masked + EPS)
        gate = jnp.where(lane < E, gate, 0.0)

        gout_v[...] = gate
        sv = jnp.float32(0.0)
        sv2 = jnp.float32(0.0)
        m = jnp.float32(0.0)
        for s in range(E):
            gs = gate[s]
            sv = sv + gs
            sv2 = sv2 + gs * gs
            m = m + jnp.where(gs > 0.0, 1.0, 0.0)
        inv_n = 1.0 / N_TOT
        inv_n1 = 1.0 / (N_TOT - 1.0)
        sv_v = jnp.broadcast_to(sv, (16,))
        sv2_v = jnp.broadcast_to(sv2, (16,))
        m_v = jnp.broadcast_to(m, (16,))
        mean_i = sv_v * inv_n
        var_i = (sv2_v - sv_v * sv_v * inv_n) * inv_n1
        loss_i = var_i / (mean_i * mean_i + 1e-10)
        mean_l = m_v * inv_n
        var_l = (m_v - m_v * m_v * inv_n) * inv_n1
        loss_l = var_l / (mean_l * mean_l + 1e-10)

        loss_v[...] = loss_i + loss_l
        pltpu.sync_copy(gout_v, gate_hbm)
        pltpu.sync_copy(loss_v, loss_hbm)


def _ffn_body(x_ref, w1_ref, b1_ref, w2_ref, b2_ref, y8_ref, outz_ref):
    i = pl.program_id(0)
    h = jnp.dot(x_ref[...], w1_ref[0], preferred_element_type=jnp.float32)
    h = jax.nn.gelu(h + b1_ref[0:1])
    part = jnp.dot(h, w2_ref[0], preferred_element_type=jnp.float32)

    @pl.when(i == 0)
    def _():
        y8_ref[...] = part

    @pl.when(i > 0)
    def _():
        y8_ref[...] += part

    @pl.when(i == GRID_H - 1)
    def _():
        y8_ref[...] += b2_ref[0:1]

    outz_ref[...] = jnp.zeros((1, S // GRID_H, D), jnp.float32)


def _ffn(x, W1, b1, W2, b2):
    return pl.pallas_call(
        _ffn_body,
        grid=(GRID_H,),
        in_specs=[
            pl.BlockSpec((E, D), lambda i: (0, 0)),
            pl.BlockSpec((1, D, H_BLK), lambda i: (0, 0, i)),
            pl.BlockSpec((E, H_BLK), lambda i: (0, i)),
            pl.BlockSpec((1, H_BLK, D), lambda i: (0, i, 0)),
            pl.BlockSpec((E, D), lambda i: (0, 0)),
        ],
        out_specs=[
            pl.BlockSpec((E, D), lambda i: (0, 0)),
            pl.BlockSpec((1, S // GRID_H, D), lambda i: (0, i, 0)),
        ],
        out_shape=[
            jax.ShapeDtypeStruct((E, D), jnp.float32),
            jax.ShapeDtypeStruct((1, S, D), jnp.float32),
        ],
    )(x, W1, b1, W2, b2)


def _finalize_body(outz_ref, y8_ref, g16_ref, out_ref):
    del outz_ref
    g = g16_ref[...][0:E]
    out_ref[...] = (y8_ref[...] * g[:, None])[None]


def _finalize(outz, y8, g16):
    return pl.pallas_call(
        _finalize_body,
        grid=(1,),
        in_specs=[
            pl.BlockSpec(memory_space=pl.ANY),
            pl.BlockSpec((E, D), lambda i: (0, 0)),
            pl.BlockSpec((16,), lambda i: (0,)),
        ],
        out_specs=pl.BlockSpec((1, E, D), lambda i: (0, 0, 0)),
        out_shape=jax.ShapeDtypeStruct((1, S, D), jnp.float32),
        input_output_aliases={0: 0},
    )(outz, y8, g16)


def kernel(x, w_gate, b_gate, W1, b1, W2, b2):
    x2 = x.reshape(S, D)
    logits_t = _gate_logits(x2, w_gate, b_gate)
    gate16, loss16 = _make_sc_routing()(logits_t)
    y8, outz = _ffn(x2, W1, b1, W2, b2)
    out = _finalize(outz, y8, gate16)
    return out, loss16[0]


# speedup vs baseline: 1.0464x; 1.0464x over previous
import functools

import jax
import jax.numpy as jnp
from jax import lax
from jax.experimental import pallas as pl
from jax.experimental.pallas import tpu as pltpu
from jax.experimental.pallas import tpu_sc as plsc

S = 2048
D = 768
E = 8
H = 3072
EPS = 1e-6
N_TOT = float(S * E)

GRID_L = 2
S_BLK = S // GRID_L
GRID_H = 2
H_BLK = H // GRID_H
GRID_Z = 4


def _gate_logits_body(x_ref, wg_ref, bg_ref, out_ref):
    h = jnp.dot(x_ref[...], wg_ref[...], preferred_element_type=jnp.float32)
    out_ref[...] = (h + bg_ref[...][None, :]).T


def _gate_logits(x, w_gate, b_gate_row):
    return pl.pallas_call(
        _gate_logits_body,
        grid=(GRID_L,),
        in_specs=[
            pl.BlockSpec((S_BLK, D), lambda i: (i, 0)),
            pl.BlockSpec((D, E), lambda i: (0, 0)),
            pl.BlockSpec((E,), lambda i: (0,)),
        ],
        out_specs=pl.BlockSpec((E, S_BLK), lambda i: (0, i)),
        out_shape=jax.ShapeDtypeStruct((E, S), jnp.float32),
    )(x, w_gate, b_gate_row)


@functools.lru_cache(maxsize=1)
def _make_sc_routing():
    mesh = plsc.VectorSubcoreMesh(core_axis_name="c", subcore_axis_name="s", num_cores=1)
    return functools.partial(
        pl.kernel,
        out_type=(
            jax.ShapeDtypeStruct((16,), jnp.float32),
            jax.ShapeDtypeStruct((16,), jnp.float32),
        ),
        mesh=mesh,
        scratch_types=[
            pltpu.VMEM((E, S), jnp.float32),
            pltpu.VMEM((16,), jnp.float32),
            pltpu.VMEM((16,), jnp.float32),
        ],
    )(_sc_routing_body)


def _sc_routing_body(logits_hbm, gate_hbm, loss_hbm, lg_v, gout_v, loss_v):
    cid = lax.axis_index("c")
    sid = lax.axis_index("s")

    @pl.when((cid == 0) & (sid == 0))
    def _():
        pltpu.sync_copy(logits_hbm, lg_v)

        def body(i, cnts):
            off = i * 16
            vs = [lg_v[e, pl.ds(off, 16)] for e in range(E)]
            mx = vs[0]
            for e in range(1, E):
                mx = jnp.maximum(mx, vs[e])
            return tuple(
                cnts[e] + jnp.where(vs[e] >= mx, 1.0, 0.0) for e in range(E)
            )

        init = tuple(jnp.zeros((16,), jnp.float32) for _ in range(E))
        cnts = lax.fori_loop(0, S // 16, body, init)

        lane = lax.iota(jnp.int32, 16)
        member = jnp.zeros((16,), jnp.float32)
        for e in range(E):
            tot = cnts[e][0]
            for l in range(1, 16):
                tot = tot + cnts[e][l]
            flag = jnp.where(tot > 0.0, 1.0, 0.0)
            member = member + jnp.where(lane == e, flag, 0.0)

        v0 = [lg_v[e, pl.ds(0, 16)] for e in range(E)]
        mx = v0[0]
        for e in range(1, E):
            mx = jnp.maximum(mx, v0[e])
        den = jnp.zeros((16,), jnp.float32)
        for e in range(E):
            den = den + jnp.exp(v0[e] - mx)
        g = jnp.exp(v0[0] - mx) / den

        masked = member * g
        gate = masked / (masked + EPS)
        gate = jnp.where(lane < E, gate, 0.0)

        gout_v[...] = gate
        sv = jnp.float32(0.0)
        sv2 = jnp.float32(0.0)
        m = jnp.float32(0.0)
        for s in range(E):
            gs = gate[s]
            sv = sv + gs
            sv2 = sv2 + gs * gs
            m = m + jnp.where(gs > 0.0, 1.0, 0.0)
        inv_n = 1.0 / N_TOT
        inv_n1 = 1.0 / (N_TOT - 1.0)
        sv_v = jnp.broadcast_to(sv, (16,))
        sv2_v = jnp.broadcast_to(sv2, (16,))
        m_v = jnp.broadcast_to(m, (16,))
        mean_i = sv_v * inv_n
        var_i = (sv2_v - sv_v * sv_v * inv_n) * inv_n1
        loss_i = var_i / (mean_i * mean_i + 1e-10)
        mean_l = m_v * inv_n
        var_l = (m_v - m_v * m_v * inv_n) * inv_n1
        loss_l = var_l / (mean_l * mean_l + 1e-10)

        loss_v[...] = loss_i + loss_l
        pltpu.sync_copy(gout_v, gate_hbm)
        pltpu.sync_copy(loss_v, loss_hbm)


def _ffn_body(x_ref, w1_ref, b1_ref, w2_ref, b2_ref, y8_ref, outz_ref):
    i = pl.program_id(0)
    h = jnp.dot(x_ref[...], w1_ref[0], preferred_element_type=jnp.float32)
    h = jax.nn.gelu(h + b1_ref[0:1])
    part = jnp.dot(h, w2_ref[0], preferred_element_type=jnp.float32)

    @pl.when(i == 0)
    def _():
        y8_ref[...] = part

    @pl.when(i > 0)
    def _():
        y8_ref[...] += part

    @pl.when(i == GRID_H - 1)
    def _():
        y8_ref[...] += b2_ref[0:1]

    outz_ref[...] = jnp.zeros((1, S // GRID_H, D), jnp.float32)


def _ffn(x, W1, b1, W2, b2):
    return pl.pallas_call(
        _ffn_body,
        grid=(GRID_H,),
        in_specs=[
            pl.BlockSpec((E, D), lambda i: (0, 0)),
            pl.BlockSpec((1, D, H_BLK), lambda i: (0, 0, i)),
            pl.BlockSpec((E, H_BLK), lambda i: (0, i)),
            pl.BlockSpec((1, H_BLK, D), lambda i: (0, i, 0)),
            pl.BlockSpec((E, D), lambda i: (0, 0)),
        ],
        out_specs=[
            pl.BlockSpec((E, D), lambda i: (0, 0)),
            pl.BlockSpec((1, S // GRID_H, D), lambda i: (0, i, 0)),
        ],
        out_shape=[
            jax.ShapeDtypeStruct((E, D), jnp.float32),
            jax.ShapeDtypeStruct((1, S, D), jnp.float32),
        ],
    )(x, W1, b1, W2, b2)


def _finalize_body(outz_ref, y8_ref, g16_ref, out_ref):
    del outz_ref
    g = g16_ref[...][0:E]
    out_ref[...] = (y8_ref[...] * g[:, None])[None]


def _finalize(outz, y8, g16):
    return pl.pallas_call(
        _finalize_body,
        grid=(1,),
        in_specs=[
            pl.BlockSpec(memory_space=pl.ANY),
            pl.BlockSpec((E, D), lambda i: (0, 0)),
            pl.BlockSpec((16,), lambda i: (0,)),
        ],
        out_specs=pl.BlockSpec((1, E, D), lambda i: (0, 0, 0)),
        out_shape=jax.ShapeDtypeStruct((1, S, D), jnp.float32),
        input_output_aliases={0: 0},
    )(outz, y8, g16)


def kernel(x, w_gate, b_gate, W1, b1, W2, b2):
    x2 = x.reshape(S, D)
    logits_t = _gate_logits(x2, w_gate, b_gate)
    gate16, loss16 = _make_sc_routing()(logits_t)
    y8, outz = _ffn(x2, W1, b1, W2, b2)
    out = _finalize(outz, y8, gate16)
    return out, loss16[0]
